# Initial kernel scaffold; baseline (speedup 1.0000x reference)
import functools

import jax
import jax.numpy as jnp
from jax import lax
from jax.experimental import pallas as pl
from jax.experimental.pallas import tpu as pltpu
from jax.experimental.pallas import tpu_sc as plsc

VOCAB = 1000000
EMBED = 64
MAX_LEN = 200
BATCH = 4096
PAD_IDX = 0
EPS = 1e-8

NUM_CORES = 2
NUM_SUBCORES = 16
LANES = 16
NW = NUM_CORES * NUM_SUBCORES
B_PER_W = BATCH // NW
BGROUPS = B_PER_W // LANES
LCHUNK = 40
NLC = MAX_LEN // LCHUNK
ROWS = LANES * LCHUNK
INV_EMBED = 1.0 / EMBED


def _rsqrt(z):
    i = plsc.bitcast(z, jnp.int32)
    y = plsc.bitcast(jnp.int32(0x5F3759DF) - (i >> 1), jnp.float32)
    for _ in range(3):
        y = y * (1.5 - 0.5 * z * y * y)
    return y


def _make_kernel():
    mesh = plsc.VectorSubcoreMesh(core_axis_name="c", subcore_axis_name="s")

    @functools.partial(
        pl.kernel,
        mesh=mesh,
        out_type=jax.ShapeDtypeStruct((BATCH, MAX_LEN, EMBED), jnp.float32),
        scratch_types=[
            pltpu.VMEM((ROWS,), jnp.int32),
            pltpu.VMEM((ROWS, EMBED), jnp.float32),
            pltpu.VMEM((MAX_LEN, EMBED), jnp.float32),
            pltpu.VMEM((EMBED,), jnp.float32),
            pltpu.VMEM((EMBED,), jnp.float32),
            pltpu.SemaphoreType.DMA,
        ],
    )
    def emb_kernel(tokens, w_word, w_pos, gamma, beta, out,
                   idx_v, rows_v, pos_v, gamma_v, beta_v, sem):
        wid = lax.axis_index("s") * NUM_CORES + lax.axis_index("c")
        lane = lax.iota(jnp.int32, LANES)

        pltpu.sync_copy(w_pos, pos_v)
        pltpu.sync_copy(gamma, gamma_v)
        pltpu.sync_copy(beta, beta_v)

        def chunk_body(ci, _):
            bg = ci // NLC
            lc = ci % NLC
            b0 = wid * B_PER_W + bg * LANES
            l0 = lc * LCHUNK

            def load_ids(i, carry):
                pltpu.sync_copy(
                    tokens.at[b0 + i, pl.ds(l0, LCHUNK)],
                    idx_v.at[pl.ds(pl.multiple_of(i * LCHUNK, 8), LCHUNK)],
                )
                return carry
            lax.fori_loop(0, LANES, load_ids, 0)

            pltpu.async_copy(w_word.at[idx_v], rows_v, sem).wait()

            def l_body(l, carry):
                row_idx = lane * LCHUNK + l

                def p1(db, acc_carry):
                    acc, acc2 = acc_carry
                    for j in range(8):
                        d = db * 8 + j
                        col = jnp.full((LANES,), d, dtype=jnp.int32)
                        v = plsc.load_gather(rows_v, [row_idx, col])
                        x = v + pos_v[l0 + l, d]
                        acc = acc + x
                        acc2 = acc2 + x * x
                        plsc.store_scatter(rows_v, [row_idx, col], x)
                    return acc, acc2

                zero = jnp.zeros((LANES,), jnp.float32)
                acc, acc2 = lax.fori_loop(0, EMBED // 8, p1, (zero, zero))

                mean = acc * INV_EMBED
                var = acc2 * INV_EMBED - mean * mean
                rstd = _rsqrt(var + EPS)
                tok = plsc.load_gather(idx_v, [row_idx])
                maskf = jnp.where(tok != PAD_IDX, 1.0, 0.0).astype(jnp.float32)
                a = rstd * maskf

                def p2(db, inner_carry):
                    for j in range(8):
                        d = db * 8 + j
                        col = jnp.full((LANES,), d, dtype=jnp.int32)
                        x = plsc.load_gather(rows_v, [row_idx, col])
                        y = (x - mean) * a * gamma_v[d] + maskf * beta_v[d]
                        plsc.store_scatter(rows_v, [row_idx, col], y)
                    return inner_carry
                lax.fori_loop(0, EMBED // 8, p2, 0)
                return carry
            lax.fori_loop(0, LCHUNK, l_body, 0)

            def store_rows(i, carry):
                pltpu.sync_copy(
                    rows_v.at[pl.ds(pl.multiple_of(i * LCHUNK, 8), LCHUNK), :],
                    out.at[b0 + i, pl.ds(l0, LCHUNK), :],
                )
                return carry
            lax.fori_loop(0, LANES, store_rows, 0)
            return carry

        lax.fori_loop(0, BGROUPS * NLC, chunk_body, 0)

    return emb_kernel


_EMB_KERNEL = _make_kernel()


def kernel(tokens, W_word, W_pos, ln_gamma, ln_beta):
    return _EMB_KERNEL(tokens.astype(jnp.int32), W_word, W_pos, ln_gamma, ln_beta)


# speedup vs baseline: 1.9803x; 1.9803x over previous
import functools

import jax
import jax.numpy as jnp
from jax import lax
from jax.experimental import pallas as pl
from jax.experimental.pallas import tpu as pltpu
from jax.experimental.pallas import tpu_sc as plsc

VOCAB = 1000000
EMBED = 64
MAX_LEN = 200
BATCH = 4096
PAD_IDX = 0
EPS = 1e-8

NUM_CORES = 2
NUM_SUBCORES = 16
LANES = 16
NW = NUM_CORES * NUM_SUBCORES
B_PER_W = BATCH // NW
BGROUPS = B_PER_W // LANES
LCHUNK = 40
NLC = MAX_LEN // LCHUNK
ROWS = LANES * LCHUNK
INV_EMBED = 1.0 / EMBED


def _rsqrt(z):
    i = plsc.bitcast(z, jnp.int32)
    y = plsc.bitcast(jnp.int32(0x5F3759DF) - (i >> 1), jnp.float32)
    for _ in range(3):
        y = y * (1.5 - 0.5 * z * y * y)
    return y


def _make_kernel():
    mesh = plsc.VectorSubcoreMesh(core_axis_name="c", subcore_axis_name="s")

    @functools.partial(
        pl.kernel,
        mesh=mesh,
        compiler_params=pltpu.CompilerParams(
            needs_layout_passes=False, use_tc_tiling_on_sc=False
        ),
        out_type=jax.ShapeDtypeStruct((BATCH, MAX_LEN, EMBED), jnp.float32),
        scratch_types=[
            pltpu.VMEM((ROWS,), jnp.int32),
            pltpu.VMEM((ROWS, EMBED), jnp.float32),
            pltpu.VMEM((MAX_LEN, EMBED), jnp.float32),
            pltpu.VMEM((EMBED,), jnp.float32),
            pltpu.VMEM((EMBED,), jnp.float32),
            pltpu.SemaphoreType.DMA,
        ],
    )
    def emb_kernel(tokens, w_word, w_pos, gamma, beta, out,
                   idx_v, rows_v, pos_v, gamma_v, beta_v, sem):
        wid = lax.axis_index("s") * NUM_CORES + lax.axis_index("c")
        lane = lax.iota(jnp.int32, LANES)

        pltpu.sync_copy(w_pos, pos_v)
        pltpu.sync_copy(gamma, gamma_v)
        pltpu.sync_copy(beta, beta_v)
        g4 = [gamma_v[pl.ds(LANES * k, LANES)] for k in range(EMBED // LANES)]
        b4 = [beta_v[pl.ds(LANES * k, LANES)] for k in range(EMBED // LANES)]

        def chunk_body(ci, chunk_carry):
            bg = ci // NLC
            lc = ci % NLC
            b0 = wid * B_PER_W + bg * LANES
            l0 = lc * LCHUNK

            def load_ids(i, carry):
                src = pl.multiple_of((b0 + i) * MAX_LEN + l0, 8)
                pltpu.sync_copy(
                    tokens.at[pl.ds(src, LCHUNK)],
                    idx_v.at[pl.ds(pl.multiple_of(i * LCHUNK, 8), LCHUNK)],
                )
                return carry
            lax.fori_loop(0, LANES, load_ids, 0)

            pltpu.async_copy(w_word.at[idx_v], rows_v, sem).wait()

            def l_body(l, carry):
                pos4 = [pos_v[l0 + l, pl.ds(LANES * k, LANES)]
                        for k in range(EMBED // LANES)]
                tokv = plsc.load_gather(idx_v, [lane * LCHUNK + l])
                maskv = jnp.where(tokv != PAD_IDX, 1.0, 0.0)

                for i in range(LANES):
                    r = i * LCHUNK + l
                    x = [rows_v[r, pl.ds(LANES * k, LANES)] + pos4[k]
                         for k in range(EMBED // LANES)]
                    s = (x[0] + x[1]) + (x[2] + x[3])
                    q = (x[0] * x[0] + x[1] * x[1]) + (x[2] * x[2] + x[3] * x[3])
                    mean = jnp.sum(s) * INV_EMBED
                    var = jnp.sum(q) * INV_EMBED - mean * mean
                    z = jnp.full((LANES,), var + EPS, dtype=jnp.float32)
                    rstd = _rsqrt(z)
                    mf = maskv[i]
                    a = rstd * mf
                    for k in range(EMBED // LANES):
                        y = (x[k] - mean) * a * g4[k] + b4[k] * mf
                        rows_v[r, pl.ds(LANES * k, LANES)] = y
                return carry
            lax.fori_loop(0, LCHUNK, l_body, 0)

            def store_rows(i, carry):
                pltpu.sync_copy(
                    rows_v.at[pl.ds(pl.multiple_of(i * LCHUNK, 8), LCHUNK), :],
                    out.at[b0 + i, pl.ds(l0, LCHUNK), :],
                )
                return carry
            lax.fori_loop(0, LANES, store_rows, 0)
            return chunk_carry

        lax.fori_loop(0, BGROUPS * NLC, chunk_body, 0)

    return emb_kernel


_EMB_KERNEL = _make_kernel()


def kernel(tokens, W_word, W_pos, ln_gamma, ln_beta):
    tok_flat = tokens.astype(jnp.int32).reshape(-1)
    return _EMB_KERNEL(tok_flat, W_word, W_pos, ln_gamma, ln_beta)


# speedup vs baseline: 2.4278x; 1.2260x over previous
import functools

import jax
import jax.numpy as jnp
from jax import lax
from jax.experimental import pallas as pl
from jax.experimental.pallas import tpu as pltpu
from jax.experimental.pallas import tpu_sc as plsc

VOCAB = 1000000
EMBED = 64
MAX_LEN = 200
BATCH = 4096
PAD_IDX = 0
EPS = 1e-8

NUM_CORES = 2
NUM_SUBCORES = 16
LANES = 16
NW = NUM_CORES * NUM_SUBCORES
B_PER_W = BATCH // NW
BGROUPS = B_PER_W // LANES
LCHUNK = 40
NLC = MAX_LEN // LCHUNK
NCHUNKS = BGROUPS * NLC
ROWS = LANES * LCHUNK
INV_EMBED = 1.0 / EMBED
NVEC = EMBED // LANES


def _rsqrt(z):
    i = plsc.bitcast(z, jnp.int32)
    y = plsc.bitcast(jnp.int32(0x5F3759DF) - (i >> 1), jnp.float32)
    for _ in range(2):
        y = y * (1.5 - 0.5 * z * y * y)
    return y


def _make_kernel():
    mesh = plsc.VectorSubcoreMesh(core_axis_name="c", subcore_axis_name="s")

    @functools.partial(
        pl.kernel,
        mesh=mesh,
        compiler_params=pltpu.CompilerParams(
            needs_layout_passes=False, use_tc_tiling_on_sc=False
        ),
        out_type=jax.ShapeDtypeStruct((BATCH, MAX_LEN, EMBED), jnp.float32),
        scratch_types=[
            pltpu.VMEM((ROWS,), jnp.int32),
            pltpu.VMEM((ROWS,), jnp.int32),
            pltpu.VMEM((ROWS, EMBED), jnp.float32),
            pltpu.VMEM((ROWS, EMBED), jnp.float32),
            pltpu.VMEM((MAX_LEN, EMBED), jnp.float32),
            pltpu.VMEM((EMBED,), jnp.float32),
            pltpu.VMEM((EMBED,), jnp.float32),
            pltpu.SemaphoreType.DMA,
            pltpu.SemaphoreType.DMA,
            pltpu.SemaphoreType.DMA,
            pltpu.SemaphoreType.DMA,
            pltpu.SemaphoreType.DMA,
        ],
    )
    def emb_kernel(tok_r, w_word, w_pos, gamma, beta, out,
                   idx0, idx1, rows0, rows1, pos_v, gamma_v, beta_v,
                   isem, gsem0, gsem1, osem0, osem1):
        wid = lax.axis_index("s") * NUM_CORES + lax.axis_index("c")
        lane = lax.iota(jnp.int32, LANES)
        idx_b = (idx0, idx1)
        rows_b = (rows0, rows1)
        gsem_b = (gsem0, gsem1)
        osem_b = (osem0, osem1)

        pltpu.sync_copy(w_pos, pos_v)
        pltpu.sync_copy(gamma, gamma_v)
        pltpu.sync_copy(beta, beta_v)
        g4 = [gamma_v[pl.ds(LANES * k, LANES)] for k in range(NVEC)]
        b4 = [beta_v[pl.ds(LANES * k, LANES)] for k in range(NVEC)]

        def tok_off(ci):
            return pl.multiple_of((wid * NCHUNKS + ci) * ROWS, 8)

        def stage_and_gather(ci, p):
            pltpu.sync_copy(tok_r.at[pl.ds(tok_off(ci), ROWS)], idx_b[p])
            pltpu.make_async_copy(
                w_word.at[idx_b[p]], rows_b[p], gsem_b[p]
            ).start()

        def wait_gather(p):
            pltpu.make_async_copy(
                w_word.at[idx_b[p]], rows_b[p], gsem_b[p]
            ).wait()

        def out_copy(ci, p, i):
            bg = ci // NLC
            lc = ci % NLC
            b0 = wid * B_PER_W + bg * LANES
            l0 = lc * LCHUNK
            return pltpu.make_async_copy(
                rows_b[p].at[pl.ds(pl.multiple_of(i * LCHUNK, 8), LCHUNK), :],
                out.at[b0 + i, pl.ds(l0, LCHUNK), :],
                osem_b[p],
            )

        def compute(ci, p):
            rows_v = rows_b[p]
            idx_v = idx_b[p]
            lc = ci % NLC
            l0 = lc * LCHUNK

            def l_body(l, carry):
                pos4 = [pos_v[l0 + l, pl.ds(LANES * k, LANES)]
                        for k in range(NVEC)]
                tokv = plsc.load_gather(idx_v, [lane * LCHUNK + l])
                maskv = jnp.where(tokv != PAD_IDX, 1.0, 0.0)

                for i in range(LANES):
                    r = i * LCHUNK + l
                    x = [rows_v[r, pl.ds(LANES * k, LANES)] + pos4[k]
                         for k in range(NVEC)]
                    s = (x[0] + x[1]) + (x[2] + x[3])
                    q = (x[0] * x[0] + x[1] * x[1]) + (x[2] * x[2] + x[3] * x[3])
                    mean = jnp.sum(s) * INV_EMBED
                    var = jnp.sum(q) * INV_EMBED - mean * mean
                    z = jnp.full((LANES,), var + EPS, dtype=jnp.float32)
                    rstd = _rsqrt(z)
                    mf = maskv[i]
                    a = rstd * mf
                    for k in range(NVEC):
                        y = (x[k] - mean) * a * g4[k] + b4[k] * mf
                        rows_v[r, pl.ds(LANES * k, LANES)] = y
                return carry
            lax.fori_loop(0, LCHUNK, l_body, 0)

        stage_and_gather(0, 0)

        def pair_body(h, carry):
            for p in (0, 1):
                ci = h * 2 + p
                wait_gather(p)

                @pl.when(ci + 1 < NCHUNKS)
                def _prep():
                    @pl.when(ci > 0)
                    def _drain():
                        for i in range(LANES):
                            out_copy(ci - 1, 1 - p, i).wait()
                    stage_and_gather(ci + 1, 1 - p)

                compute(ci, p)
                for i in range(LANES):
                    out_copy(ci, p, i).start()
            return carry

        lax.fori_loop(0, NCHUNKS // 2, pair_body, 0)

        for i in range(LANES):
            out_copy(NCHUNKS - 2, 0, i).wait()
        for i in range(LANES):
            out_copy(NCHUNKS - 1, 1, i).wait()

    return emb_kernel


_EMB_KERNEL = _make_kernel()


def kernel(tokens, W_word, W_pos, ln_gamma, ln_beta):
    tok = tokens.astype(jnp.int32)
    tok_r = (
        tok.reshape(NW, BGROUPS, LANES, NLC, LCHUNK)
        .transpose(0, 1, 3, 2, 4)
        .reshape(-1)
    )
    return _EMB_KERNEL(tok_r, W_word, W_pos, ln_gamma, ln_beta)
